# baseline (speedup 1.0000x reference)
import functools

import jax
import jax.numpy as jnp
from jax import lax
from jax.experimental import pallas as pl
from jax.experimental.pallas import tpu as pltpu
from jax.experimental.pallas import tpu_sc as plsc

L = 16


def _softplus(x):
    return jnp.maximum(x, 0.0) + jnp.log1p(jnp.exp(-jnp.abs(x)))


def _pack_halves(x):
    u = lax.bitcast_convert_type(x.astype(jnp.bfloat16), jnp.uint16)
    m = x.shape[-1] // 2
    lo = u[..., :m].astype(jnp.uint32)
    hi = u[..., m:].astype(jnp.uint32)
    return lax.bitcast_convert_type(lo | (hi << 16), jnp.int32)


def _prep_body(rel_ref, freq_ref, ent_ref, nrel_ref, dir_ref, mask_ref,
               eta_ref, w_ref, b_ref,
               cs_ref, struct_ref):
    K = ent_ref.shape[1]
    D = rel_ref.shape[1]
    ph = rel_ref[...]
    cs_ref[:, :D // 2] = _pack_halves(jnp.cos(ph))
    cs_ref[:, D // 2:] = _pack_halves(jnp.sin(ph))

    eta = _softplus(eta_ref[0, 0])
    w = _softplus(w_ref[0, 0])
    b = b_ref[0, 0]
    x = jnp.log1p(freq_ref[...])
    g = 1.0 / (1.0 + jnp.exp(-(w * x + b)))
    coeff = (1.0 - g) * eta

    mask = mask_ref[...]
    deg = jnp.maximum(jnp.sum(mask, axis=1, keepdims=True), 1.0)
    scale = coeff.reshape(-1, 1) / deg
    cc = mask * scale
    ss = cc * (1.0 - 2.0 * dir_ref[...])
    struct_ref[:, 0 * K:1 * K] = ent_ref[...]
    struct_ref[:, 1 * K:2 * K] = nrel_ref[...]
    struct_ref[:, 2 * K:3 * K] = lax.bitcast_convert_type(cc, jnp.int32)
    struct_ref[:, 3 * K:4 * K] = lax.bitcast_convert_type(ss, jnp.int32)


def _make_sc_kernel(B, N, K, D, R):
    TWO_D = 2 * D
    HD = D // 2
    try:
        info = plsc.get_sparse_core_info()
        NC, NS = info.num_cores, info.num_subcores
    except ValueError:
        NC, NS = 2, 16
    NW = NC * NS
    assert B % NW == 0
    PB = B // NW
    NCH = HD // L
    mesh = plsc.VectorSubcoreMesh(core_axis_name="c", subcore_axis_name="s",
                                  num_cores=NC, num_subcores=NS)

    @functools.partial(
        pl.kernel,
        mesh=mesh,
        out_type=jax.ShapeDtypeStruct((B, TWO_D), jnp.float32),
        compiler_params=pltpu.CompilerParams(needs_layout_passes=False),
        scratch_types=[
            pltpu.VMEM((PB,), jnp.int32),
            pltpu.VMEM((PB, 4 * K), jnp.int32),
            pltpu.VMEM((R, D), jnp.int32),
            pltpu.VMEM((D,), jnp.float32),
            pltpu.VMEM((2, K, D), jnp.int32),
            pltpu.VMEM((PB, TWO_D), jnp.float32),
            pltpu.SemaphoreType.DMA,
            pltpu.SemaphoreType.DMA,
            pltpu.SemaphoreType.DMA,
        ],
    )
    def sc_main(anchor_hbm, struct_hbm, entity_hbm, ebf_hbm,
                cs_hbm, a_hbm, out_hbm,
                aid_v, struct_v, cs_l, a_v, e_buf, out_v,
                sem0, semb0, semb1):
        wid = lax.axis_index("s") * NC + lax.axis_index("c")
        base = wid * PB
        sembs = (semb0, semb1)

        pltpu.sync_copy(anchor_hbm.at[pl.ds(base, PB)], aid_v)
        pltpu.sync_copy(a_hbm, a_v)
        cps = (
            pltpu.async_copy(cs_hbm, cs_l, sem0),
            pltpu.async_copy(struct_hbm.at[aid_v], struct_v, sem0),
            pltpu.async_copy(entity_hbm.at[aid_v], out_v, sem0),
        )
        for cp in cps:
            cp.wait()

        def issue(an, p):
            pltpu.async_copy(
                ebf_hbm.at[struct_v.at[an, pl.ds(0, K)]], e_buf.at[p],
                sembs[p])

        def drain(an, p):
            pltpu.make_async_copy(
                ebf_hbm.at[struct_v.at[an, pl.ds(0, K)]], e_buf.at[p],
                sembs[p]).wait()

        iota = lax.iota(jnp.int32, L)
        iotas = [iota + c * L for c in range(NCH)]
        iotas_s = [iota + HD + c * L for c in range(NCH)]
        av_lo = [a_v[pl.ds(c * L, L)] for c in range(NCH)]
        av_hi = [a_v[pl.ds(HD + c * L, L)] for c in range(NCH)]

        dnums = lax.GatherDimensionNumbers(
            offset_dims=(), collapsed_slice_dims=(0,), start_index_map=(0,))

        def take(v, idx):
            return lax.gather(v, idx[:, None], dnums, slice_sizes=(1,),
                              mode=lax.GatherScatterMode.PROMISE_IN_BOUNDS)

        def compute(a, p):
            a_idx = jnp.full((L,), a, jnp.int32)

            halves = []
            for h in range(K // L):
                cb16 = plsc.bitcast(struct_v[a, pl.ds(2 * K + h * L, L)],
                                    jnp.float32)
                sb16 = plsc.bitcast(struct_v[a, pl.ds(3 * K + h * L, L)],
                                    jnp.float32)
                rv16 = struct_v[a, pl.ds(K + h * L, L)]
                halves.append((cb16, sb16, rv16))

            def body(k, accs):
                idx = jnp.full((L,), k, jnp.int32)
                new = list(accs)
                for h, (cb16, sb16, rv16) in enumerate(halves):
                    cb = take(cb16, idx)
                    sb = take(sb16, idx)
                    rrow = take(rv16, idx)
                    cb_bf = plsc.pack(cb, cb,
                                      format=plsc.PackFormat.INTERLEAVED)
                    sb_bf = plsc.pack(sb, sb,
                                      format=plsc.PackFormat.INTERLEAVED)
                    ke = h * L + k
                    for c in range(NCH):
                        cv = plsc.bitcast(
                            plsc.load_gather(cs_l, [rrow, iotas[c]]),
                            jnp.bfloat16)
                        sv = plsc.bitcast(
                            plsc.load_gather(cs_l, [rrow, iotas_s[c]]),
                            jnp.bfloat16)
                        rj = plsc.bitcast(e_buf[p, ke, pl.ds(c * L, L)],
                                          jnp.bfloat16)
                        ij = plsc.bitcast(e_buf[p, ke, pl.ds(HD + c * L, L)],
                                          jnp.bfloat16)
                        ca = cv * cb_bf
                        sa = sv * sb_bf
                        new[2 * c] = new[2 * c] + (rj * ca - ij * sa)
                        new[2 * c + 1] = new[2 * c + 1] + (rj * sa + ij * ca)
                return tuple(new)

            def body2(i, accs):
                a0 = body(2 * i, accs[:2 * NCH])
                a1 = body(2 * i + 1, accs[2 * NCH:])
                return tuple(a0) + tuple(a1)

            zeros = tuple(jnp.zeros((2 * L,), jnp.bfloat16)
                          for _ in range(4 * NCH))
            accs2 = lax.fori_loop(0, L // 2, body2, zeros)
            accs = [accs2[j] + accs2[2 * NCH + j] for j in range(2 * NCH)]

            for c in range(NCH):
                re_lo, re_hi = plsc.unpack(
                    accs[2 * c], format=plsc.PackFormat.INTERLEAVED)
                im_lo, im_hi = plsc.unpack(
                    accs[2 * c + 1], format=plsc.PackFormat.INTERLEAVED)
                ce = c * L + iota
                plsc.addupdate_scatter(out_v, [a_idx, ce],
                                       av_lo[c] * re_lo)
                plsc.addupdate_scatter(out_v, [a_idx, HD + ce],
                                       av_hi[c] * re_hi)
                plsc.addupdate_scatter(out_v, [a_idx, D + ce],
                                       av_lo[c] * im_lo)
                plsc.addupdate_scatter(out_v, [a_idx, D + HD + ce],
                                       av_hi[c] * im_hi)

        issue(0, 0)
        issue(1, 1)

        def outer(i, carry):
            for p in range(2):
                a = 2 * i + p
                drain(a, p)
                nxt = a + 2

                @pl.when(nxt < PB)
                def _():
                    issue(nxt, p)

                compute(a, p)
            return carry

        lax.fori_loop(0, PB // 2, outer, 0)
        pltpu.sync_copy(out_v, out_hbm.at[pl.ds(base, PB)])

    return sc_main


def kernel(anchor_ids, nbr_ent, nbr_rel, nbr_dir, nbr_mask, freq,
           entity_embedding, relation_embedding, a, eta_raw, w_raw, b):
    N, K = nbr_ent.shape
    B = anchor_ids.shape[0]
    R, D = relation_embedding.shape

    cs_t, struct_t = pl.pallas_call(
        _prep_body,
        out_shape=[
            jax.ShapeDtypeStruct((R, D), jnp.int32),
            jax.ShapeDtypeStruct((N, 4 * K), jnp.int32),
        ],
    )(relation_embedding, freq.reshape(1, N),
      nbr_ent.astype(jnp.int32), nbr_rel.astype(jnp.int32),
      nbr_dir.astype(jnp.float32), nbr_mask.astype(jnp.float32),
      eta_raw.reshape(1, 1), w_raw.reshape(1, 1), b.reshape(1, 1))

    eb = lax.bitcast_convert_type(
        entity_embedding.astype(jnp.bfloat16), jnp.uint16)
    h = D // 2
    ebf_t = lax.bitcast_convert_type(
        jnp.concatenate(
            [eb[:, :h].astype(jnp.uint32) | (eb[:, h:D].astype(jnp.uint32)
                                             << 16),
             eb[:, D:D + h].astype(jnp.uint32) | (eb[:, D + h:]
                                                  .astype(jnp.uint32) << 16)],
            axis=1),
        jnp.int32)

    sc_main = _make_sc_kernel(B, N, K, D, R)
    return sc_main(anchor_ids.astype(jnp.int32), struct_t,
                   entity_embedding, ebf_t, cs_t, a)


# speedup vs baseline: 1.0729x; 1.0729x over previous
import functools

import jax
import jax.numpy as jnp
from jax import lax
from jax.experimental import pallas as pl
from jax.experimental.pallas import tpu as pltpu
from jax.experimental.pallas import tpu_sc as plsc

L = 16


def _softplus(x):
    return jnp.maximum(x, 0.0) + jnp.log1p(jnp.exp(-jnp.abs(x)))


def _pack_halves(x):
    u = lax.bitcast_convert_type(x.astype(jnp.bfloat16), jnp.uint16)
    m = x.shape[-1] // 2
    lo = u[..., :m].astype(jnp.uint32)
    hi = u[..., m:].astype(jnp.uint32)
    return lax.bitcast_convert_type(lo | (hi << 16), jnp.int32)


def _prep_body(rel_ref, freq_ref, ent_ref, nrel_ref, dir_ref, mask_ref,
               eta_ref, w_ref, b_ref,
               cs_ref, struct_ref):
    K = ent_ref.shape[1]
    D = rel_ref.shape[1]
    ph = rel_ref[...]
    cs_ref[:, :D // 2] = _pack_halves(jnp.cos(ph))
    cs_ref[:, D // 2:] = _pack_halves(jnp.sin(ph))

    eta = _softplus(eta_ref[0, 0])
    w = _softplus(w_ref[0, 0])
    b = b_ref[0, 0]
    x = jnp.log1p(freq_ref[...])
    g = 1.0 / (1.0 + jnp.exp(-(w * x + b)))
    coeff = (1.0 - g) * eta

    mask = mask_ref[...]
    deg = jnp.maximum(jnp.sum(mask, axis=1, keepdims=True), 1.0)
    scale = coeff.reshape(-1, 1) / deg
    cc = mask * scale
    N = mask.shape[0]
    struct_ref[:, 0 * K:1 * K] = ent_ref[...]
    struct_ref[:, 1 * K:2 * K] = (2 * nrel_ref[...]
                                  + dir_ref[...].astype(jnp.int32))
    struct_ref[:, 2 * K:3 * K] = lax.bitcast_convert_type(cc, jnp.int32)
    struct_ref[:, 3 * K:4 * K] = lax.broadcasted_iota(jnp.int32, (N, K), 0)


def _make_sc_kernel(B, N, K, D, R):
    TWO_D = 2 * D
    HD = D // 2
    try:
        info = plsc.get_sparse_core_info()
        NC, NS = info.num_cores, info.num_subcores
    except ValueError:
        NC, NS = 2, 16
    NW = NC * NS
    assert B % NW == 0
    PB = B // NW
    NCH = HD // L
    mesh = plsc.VectorSubcoreMesh(core_axis_name="c", subcore_axis_name="s",
                                  num_cores=NC, num_subcores=NS)

    @functools.partial(
        pl.kernel,
        mesh=mesh,
        out_type=jax.ShapeDtypeStruct((B, TWO_D), jnp.float32),
        compiler_params=pltpu.CompilerParams(needs_layout_passes=False),
        scratch_types=[
            pltpu.VMEM((PB,), jnp.int32),
            pltpu.VMEM((PB, 4 * K), jnp.int32),
            pltpu.VMEM((R, D), jnp.int32),
            pltpu.VMEM((D,), jnp.float32),
            pltpu.VMEM((4, K, D), jnp.int32),
            pltpu.VMEM((4, 1, TWO_D), jnp.float32),
            pltpu.VMEM((4, TWO_D), jnp.float32),
            pltpu.SemaphoreType.DMA,
            pltpu.SemaphoreType.DMA,
            pltpu.SemaphoreType.DMA,
            pltpu.SemaphoreType.DMA,
            pltpu.SemaphoreType.DMA,
            pltpu.SemaphoreType.DMA,
            pltpu.SemaphoreType.DMA,
            pltpu.SemaphoreType.DMA,
            pltpu.SemaphoreType.DMA,
            pltpu.SemaphoreType.DMA,
            pltpu.SemaphoreType.DMA,
            pltpu.SemaphoreType.DMA,
            pltpu.SemaphoreType.DMA,
        ],
    )
    def sc_main(anchor_hbm, struct_hbm, entity_hbm, ebf_hbm,
                cs_hbm, a_hbm, out_hbm,
                aid_v, struct_v, cs_l, a_v, e_buf, ei_buf, o_buf,
                sem0, semb0, semb1, semb2, semb3,
                semi0, semi1, semi2, semi3,
                semo0, semo1, semo2, semo3):
        wid = lax.axis_index("s") * NC + lax.axis_index("c")
        base = wid * PB
        sembs = (semb0, semb1, semb2, semb3)
        semis = (semi0, semi1, semi2, semi3)
        semos = (semo0, semo1, semo2, semo3)

        pltpu.sync_copy(anchor_hbm.at[pl.ds(base, PB)], aid_v)
        pltpu.sync_copy(a_hbm, a_v)
        cps = (
            pltpu.async_copy(cs_hbm, cs_l, sem0),
            pltpu.async_copy(struct_hbm.at[aid_v], struct_v, sem0),
        )
        for cp in cps:
            cp.wait()

        def issue(an, p):
            pltpu.async_copy(
                ebf_hbm.at[struct_v.at[an, pl.ds(0, K)]], e_buf.at[p],
                sembs[p])
            pltpu.async_copy(
                entity_hbm.at[struct_v.at[an, pl.ds(3 * K, 1)]],
                ei_buf.at[p], semis[p])

        def drain(an, p):
            pltpu.make_async_copy(
                ebf_hbm.at[struct_v.at[an, pl.ds(0, K)]], e_buf.at[p],
                sembs[p]).wait()
            pltpu.make_async_copy(
                entity_hbm.at[struct_v.at[an, pl.ds(3 * K, 1)]],
                ei_buf.at[p], semis[p]).wait()

        def issue_out(an, p):
            pltpu.async_copy(o_buf.at[p], out_hbm.at[base + an], semos[p])

        def drain_out(an, p):
            pltpu.make_async_copy(
                o_buf.at[p], out_hbm.at[base + an], semos[p]).wait()

        iota = lax.iota(jnp.int32, L)
        iotas = [iota + c * L for c in range(NCH)]
        iotas_s = [iota + HD + c * L for c in range(NCH)]
        av_lo = [a_v[pl.ds(c * L, L)] for c in range(NCH)]
        av_hi = [a_v[pl.ds(HD + c * L, L)] for c in range(NCH)]

        dnums = lax.GatherDimensionNumbers(
            offset_dims=(), collapsed_slice_dims=(0,), start_index_map=(0,))

        def take(v, idx):
            return lax.gather(v, idx[:, None], dnums, slice_sizes=(1,),
                              mode=lax.GatherScatterMode.PROMISE_IN_BOUNDS)

        def compute(a, p):

            halves = []
            for h in range(K // L):
                cb16 = plsc.bitcast(struct_v[a, pl.ds(2 * K + h * L, L)],
                                    jnp.float32)
                rv2 = struct_v[a, pl.ds(K + h * L, L)]
                sgn = 1.0 - 2.0 * jnp.bitwise_and(rv2, 1).astype(jnp.float32)
                sb16 = cb16 * sgn
                rv16 = lax.shift_right_logical(rv2, 1)
                halves.append((cb16, sb16, rv16))

            def body(k, accs):
                idx = jnp.full((L,), k, jnp.int32)
                new = list(accs)
                for h, (cb16, sb16, rv16) in enumerate(halves):
                    cb = take(cb16, idx)
                    sb = take(sb16, idx)
                    rrow = take(rv16, idx)
                    cb_bf = plsc.pack(cb, cb,
                                      format=plsc.PackFormat.INTERLEAVED)
                    sb_bf = plsc.pack(sb, sb,
                                      format=plsc.PackFormat.INTERLEAVED)
                    ke = h * L + k
                    for c in range(NCH):
                        cv = plsc.bitcast(
                            plsc.load_gather(cs_l, [rrow, iotas[c]]),
                            jnp.bfloat16)
                        sv = plsc.bitcast(
                            plsc.load_gather(cs_l, [rrow, iotas_s[c]]),
                            jnp.bfloat16)
                        rj = plsc.bitcast(e_buf[p, ke, pl.ds(c * L, L)],
                                          jnp.bfloat16)
                        ij = plsc.bitcast(e_buf[p, ke, pl.ds(HD + c * L, L)],
                                          jnp.bfloat16)
                        ca = cv * cb_bf
                        sa = sv * sb_bf
                        new[2 * c] = new[2 * c] + (rj * ca - ij * sa)
                        new[2 * c + 1] = new[2 * c + 1] + (rj * sa + ij * ca)
                return tuple(new)

            zeros = tuple(jnp.zeros((2 * L,), jnp.bfloat16)
                          for _ in range(2 * NCH))
            accs = lax.fori_loop(0, L, body, zeros)

            for c in range(NCH):
                re_lo, re_hi = plsc.unpack(
                    accs[2 * c], format=plsc.PackFormat.INTERLEAVED)
                im_lo, im_hi = plsc.unpack(
                    accs[2 * c + 1], format=plsc.PackFormat.INTERLEAVED)
                for off, av, val in (
                        (c * L, av_lo[c], re_lo),
                        (HD + c * L, av_hi[c], re_hi),
                        (D + c * L, av_lo[c], im_lo),
                        (D + HD + c * L, av_hi[c], im_hi)):
                    o_buf[p, pl.ds(off, L)] = (
                        ei_buf[p, 0, pl.ds(off, L)] + av * val)

        NB = 4
        for p in range(NB):
            issue(p, p)

        def outer(i, carry):
            for p in range(NB):
                a = NB * i + p
                drain(a, p)

                @pl.when(a >= NB)
                def _():
                    drain_out(a - NB, p)

                compute(a, p)
                issue_out(a, p)
                nxt = a + NB

                @pl.when(nxt < PB)
                def _():
                    issue(nxt, p)
            return carry

        lax.fori_loop(0, PB // NB, outer, 0)
        for p in range(NB):
            drain_out(PB - NB + p, p)

    return sc_main


def kernel(anchor_ids, nbr_ent, nbr_rel, nbr_dir, nbr_mask, freq,
           entity_embedding, relation_embedding, a, eta_raw, w_raw, b):
    N, K = nbr_ent.shape
    B = anchor_ids.shape[0]
    R, D = relation_embedding.shape

    cs_t, struct_t = pl.pallas_call(
        _prep_body,
        out_shape=[
            jax.ShapeDtypeStruct((R, D), jnp.int32),
            jax.ShapeDtypeStruct((N, 4 * K), jnp.int32),
        ],
    )(relation_embedding, freq.reshape(1, N),
      nbr_ent.astype(jnp.int32), nbr_rel.astype(jnp.int32),
      nbr_dir.astype(jnp.float32), nbr_mask.astype(jnp.float32),
      eta_raw.reshape(1, 1), w_raw.reshape(1, 1), b.reshape(1, 1))

    eb = lax.bitcast_convert_type(
        entity_embedding.astype(jnp.bfloat16), jnp.uint16)
    h = D // 2
    ebf_t = lax.bitcast_convert_type(
        jnp.concatenate(
            [eb[:, :h].astype(jnp.uint32) | (eb[:, h:D].astype(jnp.uint32)
                                             << 16),
             eb[:, D:D + h].astype(jnp.uint32) | (eb[:, D + h:]
                                                  .astype(jnp.uint32) << 16)],
            axis=1),
        jnp.int32)

    sc_main = _make_sc_kernel(B, N, K, D, R)
    return sc_main(anchor_ids.astype(jnp.int32), struct_t,
                   entity_embedding, ebf_t, cs_t, a)
